# baseline (speedup 1.0000x reference)
import functools

import jax
import jax.numpy as jnp
from jax import lax
from jax.experimental import pallas as pl
from jax.experimental.pallas import tpu as pltpu
from jax.experimental.pallas import tpu_sc as plsc

B = 4096
N_CAT = 26
VOCAB = 100000
EMB = 16
MULTI_LEN = 20
MULTI_EMB = 16
N_NUM = 13
NUM_LAYER = 64
D1, D2, D3 = 256, 128, 64
D_CAT = N_CAT * EMB
D_MULTI = MULTI_LEN * MULTI_EMB

NC = 2
NS = 16
NW = NC * NS

CAT_ROWS = B * N_CAT
MULTI_ROWS = B * MULTI_LEN
CPW = CAT_ROWS // NW
MPW = MULTI_ROWS // NW
CHUNK = 128
C_CHUNKS = CPW // CHUNK
M_CHUNKS = MPW // CHUNK


def _sc_gather(cat_tab_flat, cat_idx, multi_tab, multi_idx):
    mesh = plsc.VectorSubcoreMesh(core_axis_name="c", subcore_axis_name="s")

    @functools.partial(
        pl.kernel,
        mesh=mesh,
        compiler_params=pltpu.CompilerParams(use_tc_tiling_on_sc=False),
        out_type=[
            jax.ShapeDtypeStruct((CAT_ROWS, EMB), jnp.float32),
            jax.ShapeDtypeStruct((MULTI_ROWS, EMB), jnp.float32),
        ],
        scratch_types=[
            pltpu.VMEM((C_CHUNKS, CHUNK), jnp.int32),
            pltpu.VMEM((CPW, EMB), jnp.float32),
            pltpu.VMEM((M_CHUNKS, CHUNK), jnp.int32),
            pltpu.VMEM((MPW, EMB), jnp.float32),
            pltpu.SemaphoreType.DMA,
            pltpu.SemaphoreType.DMA,
        ],
    )
    def k(cat_tab_hbm, cat_idx_hbm, multi_tab_hbm, multi_idx_hbm,
          cat_out, multi_out, cidx_v, crows_v, midx_v, mrows_v, csem, msem):
        wid = lax.axis_index("s") * NC + lax.axis_index("c")
        cbase = wid * CPW
        mbase = wid * MPW
        pltpu.sync_copy(cat_idx_hbm.at[wid], cidx_v)
        pltpu.sync_copy(multi_idx_hbm.at[wid], midx_v)
        copies = []
        for j in range(C_CHUNKS):
            copies.append(pltpu.async_copy(
                cat_tab_hbm.at[cidx_v.at[j]],
                crows_v.at[pl.ds(j * CHUNK, CHUNK)], csem))
        for j in range(M_CHUNKS):
            copies.append(pltpu.async_copy(
                multi_tab_hbm.at[midx_v.at[j]],
                mrows_v.at[pl.ds(j * CHUNK, CHUNK)], msem))
        for c in copies:
            c.wait()
        pltpu.sync_copy(crows_v, cat_out.at[pl.ds(cbase, CPW)])
        pltpu.sync_copy(mrows_v, multi_out.at[pl.ds(mbase, MPW)])

    return k(cat_tab_flat, cat_idx, multi_tab, multi_idx)


BLK = 512


def _tower_body(cat_ref, multi_ref, num_ref, g_ref, be_ref, mu_ref, va_ref,
                Wn_ref, bn_ref, W1_ref, b1_ref, W2_ref, b2_ref, W3_ref, b3_ref,
                out_ref):
    f32 = jnp.float32
    scale = g_ref[...] * lax.rsqrt(va_ref[...] + 1e-3)
    bn = (num_ref[...] - mu_ref[...]) * scale + be_ref[...]
    num_out = jnp.maximum(
        jnp.dot(bn, Wn_ref[...], preferred_element_type=f32) + bn_ref[...], 0.0)
    h1 = jnp.dot(cat_ref[...], W1_ref[0:D_CAT, :], preferred_element_type=f32)
    h1 = h1 + jnp.dot(multi_ref[...], W1_ref[D_CAT:D_CAT + D_MULTI, :],
                      preferred_element_type=f32)
    h1 = h1 + jnp.dot(num_out, W1_ref[D_CAT + D_MULTI:, :],
                      preferred_element_type=f32)
    h1 = jnp.maximum(h1 + b1_ref[...], 0.0)
    h2 = jnp.maximum(
        jnp.dot(h1, W2_ref[...], preferred_element_type=f32) + b2_ref[...], 0.0)
    y = jnp.dot(h2, W3_ref[...], preferred_element_type=f32) + b3_ref[...]
    denom = jnp.sqrt(jnp.maximum(jnp.sum(y * y, axis=-1, keepdims=True), 1e-12))
    out_ref[...] = y / denom + 1e-7


def _tower(cat_emb, multi_emb, numerical, bn_gamma, bn_beta, bn_mean, bn_var,
           W_num, b_num, W1, b1, W2, b2, W3, b3):
    grid = (B // BLK,)
    row_spec = lambda d: pl.BlockSpec((BLK, d), lambda i: (i, 0))
    full = lambda a: pl.BlockSpec(a.shape, lambda i: (0,) * a.ndim)
    return pl.pallas_call(
        _tower_body,
        grid=grid,
        in_specs=[
            row_spec(D_CAT), row_spec(D_MULTI), row_spec(N_NUM),
            full(bn_gamma), full(bn_beta), full(bn_mean), full(bn_var),
            full(W_num), full(b_num), full(W1), full(b1), full(W2), full(b2),
            full(W3), full(b3),
        ],
        out_specs=row_spec(D3),
        out_shape=jax.ShapeDtypeStruct((B, D3), jnp.float32),
    )(cat_emb, multi_emb, numerical, bn_gamma, bn_beta, bn_mean, bn_var,
      W_num, b_num, W1, b1, W2, b2, W3, b3)


def kernel(cat_indices, multi_indices, numerical, cat_tables, multi_table,
           bn_gamma, bn_beta, bn_mean, bn_var, W_num, b_num,
           W1, b1, W2, b2, W3, b3):
    offsets = (jnp.arange(N_CAT, dtype=jnp.int32) * (VOCAB + 1))[None, :]
    cat_idx = (cat_indices.astype(jnp.int32) + offsets).reshape(
        NW, C_CHUNKS, CHUNK)
    multi_idx = multi_indices.astype(jnp.int32).reshape(NW, M_CHUNKS, CHUNK)
    cat_tab_flat = cat_tables.reshape(N_CAT * (VOCAB + 1), EMB)

    cat_rows, multi_rows = _sc_gather(cat_tab_flat, cat_idx, multi_table,
                                      multi_idx)
    cat_emb = cat_rows.reshape(B, D_CAT)
    multi_emb = multi_rows.reshape(B, D_MULTI)

    two_d = lambda a: a.reshape(1, -1)
    return _tower(cat_emb, multi_emb, numerical,
                  two_d(bn_gamma), two_d(bn_beta), two_d(bn_mean),
                  two_d(bn_var), W_num, two_d(b_num),
                  W1, two_d(b1), W2, two_d(b2), W3, two_d(b3))


# speedup vs baseline: 1.9658x; 1.9658x over previous
import functools

import jax
import jax.numpy as jnp
from jax import lax
from jax.experimental import pallas as pl
from jax.experimental.pallas import tpu as pltpu
from jax.experimental.pallas import tpu_sc as plsc

B = 4096
N_CAT = 26
VOCAB = 100000
EMB = 16
MULTI_LEN = 20
MULTI_EMB = 16
N_NUM = 13
NUM_LAYER = 64
D1, D2, D3 = 256, 128, 64
D_CAT = N_CAT * EMB
D_MULTI = MULTI_LEN * MULTI_EMB

NC = 2
NS = 16
NW = NC * NS

BPW = B // NW
CPW = BPW * N_CAT
MULTI_ROWS = B * MULTI_LEN
MPW = MULTI_ROWS // NW
CHUNK = 128
M_CHUNKS = MPW // CHUNK


def _sc_gather(cat_tables, cat_idx, multi_tab, multi_idx):
    mesh = plsc.VectorSubcoreMesh(core_axis_name="c", subcore_axis_name="s")

    @functools.partial(
        pl.kernel,
        mesh=mesh,
        compiler_params=pltpu.CompilerParams(use_tc_tiling_on_sc=False),
        out_type=[
            jax.ShapeDtypeStruct((NW, CPW, EMB), jnp.float32),
            jax.ShapeDtypeStruct((MULTI_ROWS, EMB), jnp.float32),
        ],
        scratch_types=[
            pltpu.VMEM((N_CAT, BPW), jnp.int32),
            pltpu.VMEM((CPW, EMB), jnp.float32),
            pltpu.VMEM((M_CHUNKS, CHUNK), jnp.int32),
            pltpu.VMEM((MPW, EMB), jnp.float32),
            pltpu.SemaphoreType.DMA,
            pltpu.SemaphoreType.DMA,
        ],
    )
    def k(cat_tab_hbm, cat_idx_hbm, multi_tab_hbm, multi_idx_hbm,
          cat_out, multi_out, cidx_v, crows_v, midx_v, mrows_v, csem, msem):
        wid = lax.axis_index("s") * NC + lax.axis_index("c")
        mbase = wid * MPW
        pltpu.sync_copy(cat_idx_hbm.at[wid], cidx_v)
        pltpu.sync_copy(multi_idx_hbm.at[wid], midx_v)
        copies = []
        for f in range(N_CAT):
            copies.append(pltpu.async_copy(
                cat_tab_hbm.at[f].at[cidx_v.at[f]],
                crows_v.at[pl.ds(f * BPW, BPW)], csem))
        for j in range(M_CHUNKS):
            copies.append(pltpu.async_copy(
                multi_tab_hbm.at[midx_v.at[j]],
                mrows_v.at[pl.ds(j * CHUNK, CHUNK)], msem))
        for c in copies:
            c.wait()
        pltpu.sync_copy(crows_v, cat_out.at[wid])
        pltpu.sync_copy(mrows_v, multi_out.at[pl.ds(mbase, MPW)])

    return k(cat_tables, cat_idx, multi_tab, multi_idx)


BLK = 512
WPB = BLK // BPW


def _tower_body(cat_ref, multi_ref, num_ref, g_ref, be_ref, mu_ref, va_ref,
                Wn_ref, bn_ref, W1_ref, b1_ref, W2_ref, b2_ref, W3_ref, b3_ref,
                out_ref):
    f32 = jnp.float32
    scale = g_ref[...] * lax.rsqrt(va_ref[...] + 1e-3)
    bn = (num_ref[...] - mu_ref[...]) * scale + be_ref[...]
    num_out = jnp.maximum(
        jnp.dot(bn, Wn_ref[...], preferred_element_type=f32) + bn_ref[...], 0.0)
    h1 = jnp.dot(multi_ref[...], W1_ref[D_CAT:D_CAT + D_MULTI, :],
                 preferred_element_type=f32)
    h1 = h1 + jnp.dot(num_out, W1_ref[D_CAT + D_MULTI:, :],
                      preferred_element_type=f32)
    for f in range(N_CAT):
        xf = cat_ref[:, f, :, :].reshape(BLK, EMB)
        h1 = h1 + jnp.dot(xf, W1_ref[pl.ds(f * EMB, EMB), :],
                          preferred_element_type=f32)
    h1 = jnp.maximum(h1 + b1_ref[...], 0.0)
    h2 = jnp.maximum(
        jnp.dot(h1, W2_ref[...], preferred_element_type=f32) + b2_ref[...], 0.0)
    y = jnp.dot(h2, W3_ref[...], preferred_element_type=f32) + b3_ref[...]
    denom = jnp.sqrt(jnp.maximum(jnp.sum(y * y, axis=-1, keepdims=True), 1e-12))
    out_ref[...] = y / denom + 1e-7


def _tower(cat_g, multi_emb, numerical, bn_gamma, bn_beta, bn_mean, bn_var,
           W_num, b_num, W1, b1, W2, b2, W3, b3):
    grid = (B // BLK,)
    row_spec = lambda d: pl.BlockSpec((BLK, d), lambda i: (i, 0))
    full = lambda a: pl.BlockSpec(a.shape, lambda i: (0,) * a.ndim)
    cat_spec = pl.BlockSpec((WPB, N_CAT, BPW, EMB), lambda i: (i, 0, 0, 0))
    return pl.pallas_call(
        _tower_body,
        grid=grid,
        in_specs=[
            cat_spec, row_spec(D_MULTI), row_spec(N_NUM),
            full(bn_gamma), full(bn_beta), full(bn_mean), full(bn_var),
            full(W_num), full(b_num), full(W1), full(b1), full(W2), full(b2),
            full(W3), full(b3),
        ],
        out_specs=row_spec(D3),
        out_shape=jax.ShapeDtypeStruct((B, D3), jnp.float32),
    )(cat_g, multi_emb, numerical, bn_gamma, bn_beta, bn_mean, bn_var,
      W_num, b_num, W1, b1, W2, b2, W3, b3)


def kernel(cat_indices, multi_indices, numerical, cat_tables, multi_table,
           bn_gamma, bn_beta, bn_mean, bn_var, W_num, b_num,
           W1, b1, W2, b2, W3, b3):
    cat_idx = cat_indices.astype(jnp.int32).reshape(
        NW, BPW, N_CAT).transpose(0, 2, 1)
    multi_idx = multi_indices.astype(jnp.int32).reshape(NW, M_CHUNKS, CHUNK)

    cat_rows, multi_rows = _sc_gather(cat_tables, cat_idx, multi_table,
                                      multi_idx)
    cat_g = cat_rows.reshape(NW, N_CAT, BPW, EMB)
    multi_emb = multi_rows.reshape(B, D_MULTI)

    two_d = lambda a: a.reshape(1, -1)
    return _tower(cat_g, multi_emb, numerical,
                  two_d(bn_gamma), two_d(bn_beta), two_d(bn_mean),
                  two_d(bn_var), W_num, two_d(b_num),
                  W1, two_d(b1), W2, two_d(b2), W3, two_d(b3))


# speedup vs baseline: 3.0415x; 1.5472x over previous
import functools

import jax
import jax.numpy as jnp
from jax import lax
from jax.experimental import pallas as pl
from jax.experimental.pallas import tpu as pltpu
from jax.experimental.pallas import tpu_sc as plsc

B = 4096
N_CAT = 26
VOCAB = 100000
EMB = 16
MULTI_LEN = 20
MULTI_EMB = 16
N_NUM = 13
NUM_LAYER = 64
D1, D2, D3 = 256, 128, 64
D_CAT = N_CAT * EMB
D_MULTI = MULTI_LEN * MULTI_EMB
V1 = VOCAB + 1

NC = 2
NS = 16
NW = NC * NS
TPW = D_CAT // NW
GL = B // 16


def _sc_gather(tabT, cat_idxT, multiT, multi_idxT):
    mesh = plsc.VectorSubcoreMesh(core_axis_name="c", subcore_axis_name="s")

    @functools.partial(
        pl.kernel,
        mesh=mesh,
        compiler_params=pltpu.CompilerParams(
            use_tc_tiling_on_sc=False, needs_layout_passes=False),
        out_type=[
            jax.ShapeDtypeStruct((D_CAT, B), jnp.float32),
            jax.ShapeDtypeStruct((D_MULTI, B), jnp.float32),
        ],
        scratch_types=[
            pltpu.VMEM((V1,), jnp.float32),
            pltpu.VMEM((B,), jnp.int32),
            pltpu.VMEM((B,), jnp.float32),
            pltpu.SemaphoreType.DMA,
        ],
    )
    def k(tabT_hbm, cat_idxT_hbm, multiT_hbm, multi_idxT_hbm,
          cat_out, multi_out, vec_v, idx_v, out_v, sem):
        wid = lax.axis_index("s") * NC + lax.axis_index("c")

        def gather_column(_):
            def body(i, carry):
                iv = idx_v[pl.ds(i * 16, 16)]
                out_v[pl.ds(i * 16, 16)] = plsc.load_gather(vec_v, [iv])
                return carry
            lax.fori_loop(0, GL, body, 0, unroll=8)

        for t in range(TPW):
            r = wid * TPW + t
            f = r // EMB
            pltpu.async_copy(tabT_hbm.at[r], vec_v, sem).wait()
            pltpu.sync_copy(cat_idxT_hbm.at[f], idx_v)
            gather_column(None)
            pltpu.sync_copy(out_v, cat_out.at[r])

        @pl.when(wid < EMB)
        def _():
            e = wid
            pltpu.async_copy(multiT_hbm.at[e], vec_v, sem).wait()
            for j in range(MULTI_LEN):
                pltpu.sync_copy(multi_idxT_hbm.at[j], idx_v)
                gather_column(None)
                pltpu.sync_copy(out_v, multi_out.at[j * EMB + e])

    return k(tabT, cat_idxT, multiT, multi_idxT)


BLK = 512


def _tower_body(catT_ref, multiT_ref, num_ref, g_ref, be_ref, mu_ref, va_ref,
                Wn_ref, bn_ref, W1_ref, b1_ref, W2_ref, b2_ref, W3_ref, b3_ref,
                out_ref):
    f32 = jnp.float32
    dn0 = (((0,), (0,)), ((), ()))
    scale = g_ref[...] * lax.rsqrt(va_ref[...] + 1e-3)
    bn = (num_ref[...] - mu_ref[...]) * scale + be_ref[...]
    num_out = jnp.maximum(
        jnp.dot(bn, Wn_ref[...], preferred_element_type=f32) + bn_ref[...], 0.0)
    h1 = lax.dot_general(catT_ref[...], W1_ref[0:D_CAT, :], dn0,
                         preferred_element_type=f32)
    h1 = h1 + lax.dot_general(multiT_ref[...],
                              W1_ref[D_CAT:D_CAT + D_MULTI, :], dn0,
                              preferred_element_type=f32)
    h1 = h1 + jnp.dot(num_out, W1_ref[D_CAT + D_MULTI:, :],
                      preferred_element_type=f32)
    h1 = jnp.maximum(h1 + b1_ref[...], 0.0)
    h2 = jnp.maximum(
        jnp.dot(h1, W2_ref[...], preferred_element_type=f32) + b2_ref[...], 0.0)
    y = jnp.dot(h2, W3_ref[...], preferred_element_type=f32) + b3_ref[...]
    denom = jnp.sqrt(jnp.maximum(jnp.sum(y * y, axis=-1, keepdims=True), 1e-12))
    out_ref[...] = y / denom + 1e-7


def _tower(catT, multiT, numerical, bn_gamma, bn_beta, bn_mean, bn_var,
           W_num, b_num, W1, b1, W2, b2, W3, b3):
    grid = (B // BLK,)
    row_spec = lambda d: pl.BlockSpec((BLK, d), lambda i: (i, 0))
    colT_spec = lambda d: pl.BlockSpec((d, BLK), lambda i: (0, i))
    full = lambda a: pl.BlockSpec(a.shape, lambda i: (0,) * a.ndim)
    return pl.pallas_call(
        _tower_body,
        grid=grid,
        in_specs=[
            colT_spec(D_CAT), colT_spec(D_MULTI), row_spec(N_NUM),
            full(bn_gamma), full(bn_beta), full(bn_mean), full(bn_var),
            full(W_num), full(b_num), full(W1), full(b1), full(W2), full(b2),
            full(W3), full(b3),
        ],
        out_specs=row_spec(D3),
        out_shape=jax.ShapeDtypeStruct((B, D3), jnp.float32),
    )(catT, multiT, numerical, bn_gamma, bn_beta, bn_mean, bn_var,
      W_num, b_num, W1, b1, W2, b2, W3, b3)


def kernel(cat_indices, multi_indices, numerical, cat_tables, multi_table,
           bn_gamma, bn_beta, bn_mean, bn_var, W_num, b_num,
           W1, b1, W2, b2, W3, b3):
    tabT = cat_tables.transpose(0, 2, 1).reshape(D_CAT, V1)
    multiT = multi_table.T
    cat_idxT = cat_indices.T.astype(jnp.int32)
    multi_idxT = multi_indices.T.astype(jnp.int32)

    catT, multiT_out = _sc_gather(tabT, cat_idxT, multiT, multi_idxT)

    two_d = lambda a: a.reshape(1, -1)
    return _tower(catT, multiT_out, numerical,
                  two_d(bn_gamma), two_d(bn_beta), two_d(bn_mean),
                  two_d(bn_var), W_num, two_d(b_num),
                  W1, two_d(b1), W2, two_d(b2), W3, two_d(b3))


# speedup vs baseline: 31.7098x; 10.4257x over previous
import functools

import jax
import jax.numpy as jnp
from jax import lax
from jax.experimental import pallas as pl
from jax.experimental.pallas import tpu as pltpu
from jax.experimental.pallas import tpu_sc as plsc

B = 4096
N_CAT = 26
VOCAB = 100000
EMB = 16
MULTI_LEN = 20
MULTI_EMB = 16
N_NUM = 13
NUM_LAYER = 64
D1, D2, D3 = 256, 128, 64
D_CAT = N_CAT * EMB
D_MULTI = MULTI_LEN * MULTI_EMB
V1 = VOCAB + 1

NC = 2
NS = 16
NW = NC * NS
TPW = D_CAT // NW
GL = B // 16


def _sc_gather(tabT, cat_idxT, multiT, multi_idxT):
    mesh = plsc.VectorSubcoreMesh(core_axis_name="c", subcore_axis_name="s")

    @functools.partial(
        pl.kernel,
        mesh=mesh,
        compiler_params=pltpu.CompilerParams(
            use_tc_tiling_on_sc=True, needs_layout_passes=False),
        out_type=[
            jax.ShapeDtypeStruct((D_CAT, B), jnp.float32),
            jax.ShapeDtypeStruct((D_MULTI, B), jnp.float32),
        ],
        scratch_types=[
            pltpu.VMEM((V1,), jnp.float32),
            pltpu.VMEM((B,), jnp.int32),
            pltpu.VMEM((B,), jnp.float32),
            pltpu.SemaphoreType.DMA,
        ],
    )
    def k(tabT_hbm, cat_idxT_hbm, multiT_hbm, multi_idxT_hbm,
          cat_out, multi_out, vec_v, idx_v, out_v, sem):
        wid = lax.axis_index("s") * NC + lax.axis_index("c")

        def gather_column(_):
            def body(i, carry):
                iv = idx_v[pl.ds(i * 16, 16)]
                out_v[pl.ds(i * 16, 16)] = plsc.load_gather(vec_v, [iv])
                return carry
            lax.fori_loop(0, GL, body, 0, unroll=8)

        for t in range(TPW):
            r = wid * TPW + t
            f = r // EMB
            pltpu.async_copy(tabT_hbm.at[r], vec_v, sem).wait()
            pltpu.sync_copy(cat_idxT_hbm.at[f], idx_v)
            gather_column(None)
            pltpu.sync_copy(out_v, cat_out.at[r])

        @pl.when(wid < EMB)
        def _():
            e = wid
            pltpu.async_copy(multiT_hbm.at[e], vec_v, sem).wait()
            for j in range(MULTI_LEN):
                pltpu.sync_copy(multi_idxT_hbm.at[j], idx_v)
                gather_column(None)
                pltpu.sync_copy(out_v, multi_out.at[j * EMB + e])

    return k(tabT, cat_idxT, multiT, multi_idxT)


BLK = 512


def _tower_body(catT_ref, multiT_ref, num_ref, g_ref, be_ref, mu_ref, va_ref,
                Wn_ref, bn_ref, W1_ref, b1_ref, W2_ref, b2_ref, W3_ref, b3_ref,
                out_ref):
    f32 = jnp.float32
    dn0 = (((0,), (0,)), ((), ()))
    scale = g_ref[...] * lax.rsqrt(va_ref[...] + 1e-3)
    bn = (num_ref[...] - mu_ref[...]) * scale + be_ref[...]
    num_out = jnp.maximum(
        jnp.dot(bn, Wn_ref[...], preferred_element_type=f32) + bn_ref[...], 0.0)
    h1 = lax.dot_general(catT_ref[...], W1_ref[0:D_CAT, :], dn0,
                         preferred_element_type=f32)
    h1 = h1 + lax.dot_general(multiT_ref[...],
                              W1_ref[D_CAT:D_CAT + D_MULTI, :], dn0,
                              preferred_element_type=f32)
    h1 = h1 + jnp.dot(num_out, W1_ref[D_CAT + D_MULTI:, :],
                      preferred_element_type=f32)
    h1 = jnp.maximum(h1 + b1_ref[...], 0.0)
    h2 = jnp.maximum(
        jnp.dot(h1, W2_ref[...], preferred_element_type=f32) + b2_ref[...], 0.0)
    y = jnp.dot(h2, W3_ref[...], preferred_element_type=f32) + b3_ref[...]
    denom = jnp.sqrt(jnp.maximum(jnp.sum(y * y, axis=-1, keepdims=True), 1e-12))
    out_ref[...] = y / denom + 1e-7


def _tower(catT, multiT, numerical, bn_gamma, bn_beta, bn_mean, bn_var,
           W_num, b_num, W1, b1, W2, b2, W3, b3):
    grid = (B // BLK,)
    row_spec = lambda d: pl.BlockSpec((BLK, d), lambda i: (i, 0))
    colT_spec = lambda d: pl.BlockSpec((d, BLK), lambda i: (0, i))
    full = lambda a: pl.BlockSpec(a.shape, lambda i: (0,) * a.ndim)
    return pl.pallas_call(
        _tower_body,
        grid=grid,
        in_specs=[
            colT_spec(D_CAT), colT_spec(D_MULTI), row_spec(N_NUM),
            full(bn_gamma), full(bn_beta), full(bn_mean), full(bn_var),
            full(W_num), full(b_num), full(W1), full(b1), full(W2), full(b2),
            full(W3), full(b3),
        ],
        out_specs=row_spec(D3),
        out_shape=jax.ShapeDtypeStruct((B, D3), jnp.float32),
    )(catT, multiT, numerical, bn_gamma, bn_beta, bn_mean, bn_var,
      W_num, b_num, W1, b1, W2, b2, W3, b3)


def kernel(cat_indices, multi_indices, numerical, cat_tables, multi_table,
           bn_gamma, bn_beta, bn_mean, bn_var, W_num, b_num,
           W1, b1, W2, b2, W3, b3):
    tabT = cat_tables.transpose(0, 2, 1).reshape(D_CAT, V1)
    multiT = multi_table.T
    cat_idxT = cat_indices.T.astype(jnp.int32)
    multi_idxT = multi_indices.T.astype(jnp.int32)

    catT, multiT_out = _sc_gather(tabT, cat_idxT, multiT, multi_idxT)

    two_d = lambda a: a.reshape(1, -1)
    return _tower(catT, multiT_out, numerical,
                  two_d(bn_gamma), two_d(bn_beta), two_d(bn_mean),
                  two_d(bn_var), W_num, two_d(b_num),
                  W1, two_d(b1), W2, two_d(b2), W3, two_d(b3))


# speedup vs baseline: 40.4641x; 1.2761x over previous
import functools

import jax
import jax.numpy as jnp
from jax import lax
from jax.experimental import pallas as pl
from jax.experimental.pallas import tpu as pltpu
from jax.experimental.pallas import tpu_sc as plsc

B = 4096
N_CAT = 26
VOCAB = 100000
EMB = 16
MULTI_LEN = 20
MULTI_EMB = 16
N_NUM = 13
NUM_LAYER = 64
D1, D2, D3 = 256, 128, 64
D_CAT = N_CAT * EMB
D_MULTI = MULTI_LEN * MULTI_EMB
V1 = VOCAB + 1

NC = 2
NS = 16
NW = NC * NS
TPW = D_CAT // NW
GL = B // 16


def _sc_gather(tabT, cat_idxT, multiT, multi_idxT):
    mesh = plsc.VectorSubcoreMesh(core_axis_name="c", subcore_axis_name="s")

    @functools.partial(
        pl.kernel,
        mesh=mesh,
        compiler_params=pltpu.CompilerParams(
            use_tc_tiling_on_sc=True, needs_layout_passes=False),
        out_type=[
            jax.ShapeDtypeStruct((D_CAT, B), jnp.float32),
            jax.ShapeDtypeStruct((D_MULTI, B), jnp.float32),
        ],
        scratch_types=[
            pltpu.VMEM((V1,), jnp.float32),
            pltpu.VMEM((B,), jnp.int32),
            pltpu.VMEM((B,), jnp.float32),
            pltpu.VMEM((B,), jnp.float32),
            pltpu.SemaphoreType.DMA,
            pltpu.SemaphoreType.DMA,
        ],
    )
    def k(tabT_hbm, cat_idxT_hbm, multiT_hbm, multi_idxT_hbm,
          cat_out, multi_out, vec_v, idx_v, out_a, out_b, sem, osem):
        wid = lax.axis_index("s") * NC + lax.axis_index("c")

        def gather_column(out_ref):
            def body(i, carry):
                iv = idx_v[pl.ds(i * 16, 16)]
                out_ref[pl.ds(i * 16, 16)] = plsc.load_gather(vec_v, [iv])
                return carry
            lax.fori_loop(0, GL, body, 0, unroll=8)

        wb = [None, None]
        out_bufs = [out_a, out_b]

        def column(src_row, idx_src, dst, slot):
            c = pltpu.async_copy(src_row, vec_v, sem)
            pltpu.sync_copy(idx_src, idx_v)
            if wb[slot] is not None:
                wb[slot].wait()
            c.wait()
            gather_column(out_bufs[slot])
            wb[slot] = pltpu.async_copy(out_bufs[slot], dst, osem)

        for t in range(TPW):
            r = wid * TPW + t
            f = r // EMB
            column(tabT_hbm.at[r], cat_idxT_hbm.at[f], cat_out.at[r], t % 2)

        e = wid % EMB
        j0 = (wid // EMB) * (MULTI_LEN // 2)
        mc = pltpu.async_copy(multiT_hbm.at[e], vec_v, sem)
        for u in range(MULTI_LEN // 2):
            j = j0 + u
            slot = (TPW + u) % 2
            pltpu.sync_copy(multi_idxT_hbm.at[j], idx_v)
            if wb[slot] is not None:
                wb[slot].wait()
            if u == 0:
                mc.wait()
            gather_column(out_bufs[slot])
            wb[slot] = pltpu.async_copy(out_bufs[slot],
                                        multi_out.at[j * EMB + e], osem)
        for w in wb:
            if w is not None:
                w.wait()

    return k(tabT, cat_idxT, multiT, multi_idxT)


BLK = 512


def _tower_body(catT_ref, multiT_ref, num_ref, g_ref, be_ref, mu_ref, va_ref,
                Wn_ref, bn_ref, W1_ref, b1_ref, W2_ref, b2_ref, W3_ref, b3_ref,
                out_ref):
    f32 = jnp.float32
    dn0 = (((0,), (0,)), ((), ()))
    scale = g_ref[...] * lax.rsqrt(va_ref[...] + 1e-3)
    bn = (num_ref[...] - mu_ref[...]) * scale + be_ref[...]
    num_out = jnp.maximum(
        jnp.dot(bn, Wn_ref[...], preferred_element_type=f32) + bn_ref[...], 0.0)
    h1 = lax.dot_general(catT_ref[...], W1_ref[0:D_CAT, :], dn0,
                         preferred_element_type=f32)
    h1 = h1 + lax.dot_general(multiT_ref[...],
                              W1_ref[D_CAT:D_CAT + D_MULTI, :], dn0,
                              preferred_element_type=f32)
    h1 = h1 + jnp.dot(num_out, W1_ref[D_CAT + D_MULTI:, :],
                      preferred_element_type=f32)
    h1 = jnp.maximum(h1 + b1_ref[...], 0.0)
    h2 = jnp.maximum(
        jnp.dot(h1, W2_ref[...], preferred_element_type=f32) + b2_ref[...], 0.0)
    y = jnp.dot(h2, W3_ref[...], preferred_element_type=f32) + b3_ref[...]
    denom = jnp.sqrt(jnp.maximum(jnp.sum(y * y, axis=-1, keepdims=True), 1e-12))
    out_ref[...] = y / denom + 1e-7


def _tower(catT, multiT, numerical, bn_gamma, bn_beta, bn_mean, bn_var,
           W_num, b_num, W1, b1, W2, b2, W3, b3):
    grid = (B // BLK,)
    row_spec = lambda d: pl.BlockSpec((BLK, d), lambda i: (i, 0))
    colT_spec = lambda d: pl.BlockSpec((d, BLK), lambda i: (0, i))
    full = lambda a: pl.BlockSpec(a.shape, lambda i: (0,) * a.ndim)
    return pl.pallas_call(
        _tower_body,
        grid=grid,
        in_specs=[
            colT_spec(D_CAT), colT_spec(D_MULTI), row_spec(N_NUM),
            full(bn_gamma), full(bn_beta), full(bn_mean), full(bn_var),
            full(W_num), full(b_num), full(W1), full(b1), full(W2), full(b2),
            full(W3), full(b3),
        ],
        out_specs=row_spec(D3),
        out_shape=jax.ShapeDtypeStruct((B, D3), jnp.float32),
    )(catT, multiT, numerical, bn_gamma, bn_beta, bn_mean, bn_var,
      W_num, b_num, W1, b1, W2, b2, W3, b3)


def kernel(cat_indices, multi_indices, numerical, cat_tables, multi_table,
           bn_gamma, bn_beta, bn_mean, bn_var, W_num, b_num,
           W1, b1, W2, b2, W3, b3):
    tabT = cat_tables.transpose(0, 2, 1).reshape(D_CAT, V1)
    multiT = multi_table.T
    cat_idxT = cat_indices.T.astype(jnp.int32)
    multi_idxT = multi_indices.T.astype(jnp.int32)

    catT, multiT_out = _sc_gather(tabT, cat_idxT, multiT, multi_idxT)

    two_d = lambda a: a.reshape(1, -1)
    return _tower(catT, multiT_out, numerical,
                  two_d(bn_gamma), two_d(bn_beta), two_d(bn_mean),
                  two_d(bn_var), W_num, two_d(b_num),
                  W1, two_d(b1), W2, two_d(b2), W3, two_d(b3))


# speedup vs baseline: 41.1546x; 1.0171x over previous
import functools

import jax
import jax.numpy as jnp
from jax import lax
from jax.experimental import pallas as pl
from jax.experimental.pallas import tpu as pltpu
from jax.experimental.pallas import tpu_sc as plsc

B = 4096
N_CAT = 26
VOCAB = 100000
EMB = 16
MULTI_LEN = 20
MULTI_EMB = 16
N_NUM = 13
NUM_LAYER = 64
D1, D2, D3 = 256, 128, 64
D_CAT = N_CAT * EMB
D_MULTI = MULTI_LEN * MULTI_EMB
V1 = VOCAB + 1

NC = 2
NS = 16
NW = NC * NS
TPW = D_CAT // NW
GL = B // 16


def _sc_gather(tabT, cat_idxT, multiT, multi_idxT):
    mesh = plsc.VectorSubcoreMesh(core_axis_name="c", subcore_axis_name="s")

    @functools.partial(
        pl.kernel,
        mesh=mesh,
        compiler_params=pltpu.CompilerParams(
            use_tc_tiling_on_sc=True, needs_layout_passes=False),
        out_type=[
            jax.ShapeDtypeStruct((D_CAT, B), jnp.float32),
            jax.ShapeDtypeStruct((D_MULTI, B), jnp.float32),
        ],
        scratch_types=[
            pltpu.VMEM((V1,), jnp.float32),
            pltpu.VMEM((B,), jnp.int32),
            pltpu.VMEM((B,), jnp.float32),
            pltpu.VMEM((B,), jnp.float32),
            pltpu.SemaphoreType.DMA,
            pltpu.SemaphoreType.DMA,
        ],
    )
    def k(tabT_hbm, cat_idxT_hbm, multiT_hbm, multi_idxT_hbm,
          cat_out, multi_out, vec_v, idx_v, out_a, out_b, sem, osem):
        wid = lax.axis_index("s") * NC + lax.axis_index("c")

        def gather_column(out_ref):
            def body(i, carry):
                iv = idx_v[pl.ds(i * 16, 16)]
                out_ref[pl.ds(i * 16, 16)] = plsc.load_gather(vec_v, [iv])
                return carry
            lax.fori_loop(0, GL, body, 0, unroll=16)

        wb = [None, None]
        out_bufs = [out_a, out_b]

        def column(src_row, idx_src, dst, slot):
            c = pltpu.async_copy(src_row, vec_v, sem)
            pltpu.sync_copy(idx_src, idx_v)
            if wb[slot] is not None:
                wb[slot].wait()
            c.wait()
            gather_column(out_bufs[slot])
            wb[slot] = pltpu.async_copy(out_bufs[slot], dst, osem)

        for t in range(TPW):
            r = wid * TPW + t
            f = r // EMB
            column(tabT_hbm.at[r], cat_idxT_hbm.at[f], cat_out.at[r], t % 2)

        e = wid % EMB
        j0 = (wid // EMB) * (MULTI_LEN // 2)
        mc = pltpu.async_copy(multiT_hbm.at[e], vec_v, sem)
        for u in range(MULTI_LEN // 2):
            j = j0 + u
            slot = (TPW + u) % 2
            pltpu.sync_copy(multi_idxT_hbm.at[j], idx_v)
            if wb[slot] is not None:
                wb[slot].wait()
            if u == 0:
                mc.wait()
            gather_column(out_bufs[slot])
            wb[slot] = pltpu.async_copy(out_bufs[slot],
                                        multi_out.at[j * EMB + e], osem)
        for w in wb:
            if w is not None:
                w.wait()

    return k(tabT, cat_idxT, multiT, multi_idxT)


BLK = 1024


def _tower_body(catT_ref, multiT_ref, num_ref, g_ref, be_ref, mu_ref, va_ref,
                Wn_ref, bn_ref, W1_ref, b1_ref, W2_ref, b2_ref, W3_ref, b3_ref,
                out_ref):
    f32 = jnp.float32
    dn0 = (((0,), (0,)), ((), ()))
    scale = g_ref[...] * lax.rsqrt(va_ref[...] + 1e-3)
    bn = (num_ref[...] - mu_ref[...]) * scale + be_ref[...]
    num_out = jnp.maximum(
        jnp.dot(bn, Wn_ref[...], preferred_element_type=f32) + bn_ref[...], 0.0)
    h1 = lax.dot_general(catT_ref[...], W1_ref[0:D_CAT, :], dn0,
                         preferred_element_type=f32)
    h1 = h1 + lax.dot_general(multiT_ref[...],
                              W1_ref[D_CAT:D_CAT + D_MULTI, :], dn0,
                              preferred_element_type=f32)
    h1 = h1 + jnp.dot(num_out, W1_ref[D_CAT + D_MULTI:, :],
                      preferred_element_type=f32)
    h1 = jnp.maximum(h1 + b1_ref[...], 0.0)
    h2 = jnp.maximum(
        jnp.dot(h1, W2_ref[...], preferred_element_type=f32) + b2_ref[...], 0.0)
    y = jnp.dot(h2, W3_ref[...], preferred_element_type=f32) + b3_ref[...]
    denom = jnp.sqrt(jnp.maximum(jnp.sum(y * y, axis=-1, keepdims=True), 1e-12))
    out_ref[...] = y / denom + 1e-7


def _tower(catT, multiT, numerical, bn_gamma, bn_beta, bn_mean, bn_var,
           W_num, b_num, W1, b1, W2, b2, W3, b3):
    grid = (B // BLK,)
    row_spec = lambda d: pl.BlockSpec((BLK, d), lambda i: (i, 0))
    colT_spec = lambda d: pl.BlockSpec((d, BLK), lambda i: (0, i))
    full = lambda a: pl.BlockSpec(a.shape, lambda i: (0,) * a.ndim)
    return pl.pallas_call(
        _tower_body,
        grid=grid,
        in_specs=[
            colT_spec(D_CAT), colT_spec(D_MULTI), row_spec(N_NUM),
            full(bn_gamma), full(bn_beta), full(bn_mean), full(bn_var),
            full(W_num), full(b_num), full(W1), full(b1), full(W2), full(b2),
            full(W3), full(b3),
        ],
        out_specs=row_spec(D3),
        out_shape=jax.ShapeDtypeStruct((B, D3), jnp.float32),
    )(catT, multiT, numerical, bn_gamma, bn_beta, bn_mean, bn_var,
      W_num, b_num, W1, b1, W2, b2, W3, b3)


def kernel(cat_indices, multi_indices, numerical, cat_tables, multi_table,
           bn_gamma, bn_beta, bn_mean, bn_var, W_num, b_num,
           W1, b1, W2, b2, W3, b3):
    tabT = cat_tables.transpose(0, 2, 1).reshape(D_CAT, V1)
    multiT = multi_table.T
    cat_idxT = cat_indices.T.astype(jnp.int32)
    multi_idxT = multi_indices.T.astype(jnp.int32)

    catT, multiT_out = _sc_gather(tabT, cat_idxT, multiT, multi_idxT)

    two_d = lambda a: a.reshape(1, -1)
    return _tower(catT, multiT_out, numerical,
                  two_d(bn_gamma), two_d(bn_beta), two_d(bn_mean),
                  two_d(bn_var), W_num, two_d(b_num),
                  W1, two_d(b1), W2, two_d(b2), W3, two_d(b3))
